# baseline (speedup 1.0000x reference)
import functools

import jax
import jax.numpy as jnp
from jax import lax
from jax.experimental import pallas as pl
from jax.experimental.pallas import tpu as pltpu
from jax.experimental.pallas import tpu_sc as plsc

N_ROWS = 50000
N_COLS = 80
CHUNK_ROWS = 80
N_CHUNKS = N_ROWS // CHUNK_ROWS
N_WORKERS = 32
MAX_CHUNKS_PER_WORKER = 20

_L1P = (6.944574124645442e-05, 0.9962619482337944, -0.46644243862756857,
        0.21866548366222538, -0.055459313742082655)
_LOG2E = 1.4426950408889634

_SIGN = -2147483648


def _poly_l1p(u):
    p = jnp.full((16,), _L1P[-1], jnp.float32)
    for c in _L1P[-2::-1]:
        p = p * u + jnp.float32(c)
    return p


def _base_sig(x):
    neg_abs = plsc.bitcast(plsc.bitcast(x, jnp.int32) | jnp.int32(_SIGN),
                           jnp.float32)
    u = jnp.exp(neg_abs)
    base = jnp.maximum(x, jnp.float32(0)) + _poly_l1p(u)
    r = jnp.float32(1) / (jnp.float32(1) + u)
    sig = r * jnp.exp(jnp.minimum(x, jnp.float32(0)))
    return base, sig


def _qfl_body(x_hbm, lbl_hbm, sco_hbm, wgt_hbm, out_hbm,
              xb0, xb1, lb0, lb1, sb0, sb1, wb0, wb1, acc_ref,
              sem0, sem1):
    core = lax.axis_index("c")
    sub = lax.axis_index("s")
    wid = sub * 2 + core
    nch = jnp.where(wid < 17, 20, 19)

    iota = lax.iota(jnp.int32, 16)
    acc_ref[...] = jnp.zeros((16,), jnp.float32)

    bufs = ((xb0, lb0, sb0, wb0, sem0), (xb1, lb1, sb1, wb1, sem1))

    def issue(n, slot):
        xb, lb, sb, wb, sem = bufs[slot]
        cid = wid + n * N_WORKERS
        r0 = cid * CHUNK_ROWS
        pltpu.async_copy(x_hbm.at[pl.ds(r0, CHUNK_ROWS)], xb, sem)
        pltpu.async_copy(lbl_hbm.at[pl.ds(r0, CHUNK_ROWS)], lb, sem)
        pltpu.async_copy(sco_hbm.at[pl.ds(r0, CHUNK_ROWS)], sb, sem)
        pltpu.async_copy(wgt_hbm.at[pl.ds(r0, CHUNK_ROWS)], wb, sem)

    def wait(slot):
        xb, lb, sb, wb, sem = bufs[slot]
        pltpu.make_async_copy(x_hbm.at[pl.ds(0, CHUNK_ROWS)], xb, sem).wait()
        pltpu.make_async_copy(lbl_hbm.at[pl.ds(0, CHUNK_ROWS)], lb, sem).wait()
        pltpu.make_async_copy(sco_hbm.at[pl.ds(0, CHUNK_ROWS)], sb, sem).wait()
        pltpu.make_async_copy(wgt_hbm.at[pl.ds(0, CHUNK_ROWS)], wb, sem).wait()

    def row_neg_sum(xb, j):
        terms = []
        for k in range(N_COLS // 16):
            x = xb[j, pl.ds(k * 16, 16)]
            base, sig = _base_sig(x)
            terms.append(base * sig * sig)
        return ((terms[0] + terms[1]) + (terms[2] + terms[3])) + terms[4]

    def process(slot):
        xb, lb, sb, wb, _ = bufs[slot]

        def row_body(i, acc):
            j = 4 * i
            tw = []
            for d in range(4):
                t = row_neg_sum(xb, j + d)
                w = plsc.load_gather(wb, [jnp.full((16,), d, jnp.int32) + j])
                tw.append(t * w)
            return acc + ((tw[0] + tw[1]) + (tw[2] + tw[3]))

        acc = lax.fori_loop(0, CHUNK_ROWS // 4, row_body,
                            jnp.zeros((16,), jnp.float32))

        for g in range(CHUNK_ROWS // 16):
            rowv = g * 16 + iota
            lbl = lb[pl.ds(g * 16, 16)]
            sco = sb[pl.ds(g * 16, 16)]
            wgt = wb[pl.ds(g * 16, 16)]
            mask = (lbl >= 0) & (lbl < N_COLS)
            safe = jnp.where(mask, lbl, 0)
            xp = plsc.load_gather(xb, [rowv, safe])
            bp, sp = _base_sig(xp)
            d = sco - sp
            corr = (bp - xp * sco) * d * d - bp * sp * sp
            acc = acc + jnp.where(mask, corr, jnp.float32(0)) * wgt
        acc_ref[...] += acc

    issue(0, 0)

    def pair_body(i, carry):
        @pl.when(2 * i + 1 < nch)
        def _():
            issue(2 * i + 1, 1)
        wait(0)
        process(0)

        @pl.when(2 * i + 2 < nch)
        def _():
            issue(2 * i + 2, 0)

        @pl.when(2 * i + 1 < nch)
        def _():
            wait(1)
            process(1)
        return carry

    lax.fori_loop(0, MAX_CHUNKS_PER_WORKER // 2, pair_body, 0)

    pltpu.sync_copy(acc_ref, out_hbm.at[wid])


@functools.partial(jax.jit, static_argnames=())
def _qfl_partials(x, lbl, sco, wgt):
    kfn = pl.kernel(
        _qfl_body,
        out_type=jax.ShapeDtypeStruct((N_WORKERS, 16), jnp.float32),
        mesh=plsc.VectorSubcoreMesh(core_axis_name="c", subcore_axis_name="s"),
        compiler_params=pltpu.CompilerParams(needs_layout_passes=False,
                                             use_tc_tiling_on_sc=True),
        scratch_types=[
            pltpu.VMEM((CHUNK_ROWS, N_COLS), jnp.float32),
            pltpu.VMEM((CHUNK_ROWS, N_COLS), jnp.float32),
            pltpu.VMEM((CHUNK_ROWS,), jnp.int32),
            pltpu.VMEM((CHUNK_ROWS,), jnp.int32),
            pltpu.VMEM((CHUNK_ROWS,), jnp.float32),
            pltpu.VMEM((CHUNK_ROWS,), jnp.float32),
            pltpu.VMEM((CHUNK_ROWS,), jnp.float32),
            pltpu.VMEM((CHUNK_ROWS,), jnp.float32),
            pltpu.VMEM((16,), jnp.float32),
            pltpu.SemaphoreType.DMA,
            pltpu.SemaphoreType.DMA,
        ],
    )
    return kfn(x, lbl, sco, wgt)


def kernel(output, label, score, weight, avg_factor):
    partials = _qfl_partials(output, label.astype(jnp.int32), score, weight)
    return partials.sum() / avg_factor


# speedup vs baseline: 1.3185x; 1.3185x over previous
import functools

import jax
import jax.numpy as jnp
from jax import lax
from jax.experimental import pallas as pl
from jax.experimental.pallas import tpu as pltpu
from jax.experimental.pallas import tpu_sc as plsc

N_ROWS = 50000
N_COLS = 80
CHUNK_ROWS = 80
N_CHUNKS = N_ROWS // CHUNK_ROWS
N_WORKERS = 32
MAX_CHUNKS_PER_WORKER = 20

_L1P = (6.944574124645442e-05, 0.9962619482337944, -0.46644243862756857,
        0.21866548366222538, -0.055459313742082655)
_LOG2E = 1.4426950408889634

_SIGN = -2147483648


def _poly_l1p(u):
    p = jnp.full((16,), _L1P[-1], jnp.float32)
    for c in _L1P[-2::-1]:
        p = p * u + jnp.float32(c)
    return p


def _base_sig(x):
    neg_abs = plsc.bitcast(plsc.bitcast(x, jnp.int32) | jnp.int32(_SIGN),
                           jnp.float32)
    u = jnp.exp(neg_abs)
    base = jnp.maximum(x, jnp.float32(0)) + _poly_l1p(u)
    r = jnp.float32(1) / (jnp.float32(1) + u)
    sig = r * jnp.exp(jnp.minimum(x, jnp.float32(0)))
    return base, sig


def _qfl_body(x_hbm, lbl_hbm, sco_hbm, wgt_hbm, out_hbm,
              xb0, xb1, lb0, lb1, sb0, sb1, wb0, wb1, acc_ref,
              sem0, sem1):
    core = lax.axis_index("c")
    sub = lax.axis_index("s")
    wid = sub * 2 + core
    nch = jnp.where(wid < 17, 20, 19)

    iota = lax.iota(jnp.int32, 16)
    acc_ref[...] = jnp.zeros((16,), jnp.float32)

    bufs = ((xb0, lb0, sb0, wb0, sem0), (xb1, lb1, sb1, wb1, sem1))

    def issue(n, slot):
        xb, lb, sb, wb, sem = bufs[slot]
        cid = wid + n * N_WORKERS
        r0 = cid * CHUNK_ROWS
        pltpu.async_copy(x_hbm.at[pl.ds(r0, CHUNK_ROWS)], xb, sem)
        pltpu.async_copy(lbl_hbm.at[pl.ds(r0, CHUNK_ROWS)], lb, sem)
        pltpu.async_copy(sco_hbm.at[pl.ds(r0, CHUNK_ROWS)], sb, sem)
        pltpu.async_copy(wgt_hbm.at[pl.ds(r0, CHUNK_ROWS)], wb, sem)

    def wait(slot):
        xb, lb, sb, wb, sem = bufs[slot]
        pltpu.make_async_copy(x_hbm.at[pl.ds(0, CHUNK_ROWS)], xb, sem).wait()
        pltpu.make_async_copy(lbl_hbm.at[pl.ds(0, CHUNK_ROWS)], lb, sem).wait()
        pltpu.make_async_copy(sco_hbm.at[pl.ds(0, CHUNK_ROWS)], sb, sem).wait()
        pltpu.make_async_copy(wgt_hbm.at[pl.ds(0, CHUNK_ROWS)], wb, sem).wait()

    def row_neg_sum(xb, j):
        terms = []
        for k in range(N_COLS // 16):
            x = xb[j, pl.ds(k * 16, 16)]
            base, sig = _base_sig(x)
            terms.append(base * sig * sig)
        return ((terms[0] + terms[1]) + (terms[2] + terms[3])) + terms[4]

    def process(slot):
        xb, lb, sb, wb, _ = bufs[slot]

        def row_body(i, acc):
            j = 2 * i
            t0 = row_neg_sum(xb, j)
            w0 = plsc.load_gather(wb, [jnp.full((16,), 0, jnp.int32) + j])
            t1 = row_neg_sum(xb, j + 1)
            w1 = plsc.load_gather(wb, [jnp.full((16,), 1, jnp.int32) + j])
            return acc + (t0 * w0 + t1 * w1)

        acc = lax.fori_loop(0, CHUNK_ROWS // 2, row_body,
                            jnp.zeros((16,), jnp.float32))

        for g in range(CHUNK_ROWS // 16):
            rowv = g * 16 + iota
            lbl = lb[pl.ds(g * 16, 16)]
            sco = sb[pl.ds(g * 16, 16)]
            wgt = wb[pl.ds(g * 16, 16)]
            mask = (lbl >= 0) & (lbl < N_COLS)
            safe = jnp.where(mask, lbl, 0)
            xp = plsc.load_gather(xb, [rowv, safe])
            bp, sp = _base_sig(xp)
            d = sco - sp
            corr = (bp - xp * sco) * d * d - bp * sp * sp
            acc = acc + jnp.where(mask, corr, jnp.float32(0)) * wgt
        acc_ref[...] += acc

    issue(0, 0)

    def pair_body(i, carry):
        @pl.when(2 * i + 1 < nch)
        def _():
            issue(2 * i + 1, 1)
        wait(0)
        process(0)

        @pl.when(2 * i + 2 < nch)
        def _():
            issue(2 * i + 2, 0)

        @pl.when(2 * i + 1 < nch)
        def _():
            wait(1)
            process(1)
        return carry

    lax.fori_loop(0, MAX_CHUNKS_PER_WORKER // 2, pair_body, 0)

    pltpu.sync_copy(acc_ref, out_hbm.at[wid])


@functools.partial(jax.jit, static_argnames=())
def _qfl_partials(x, lbl, sco, wgt):
    kfn = pl.kernel(
        _qfl_body,
        out_type=jax.ShapeDtypeStruct((N_WORKERS, 16), jnp.float32),
        mesh=plsc.VectorSubcoreMesh(core_axis_name="c", subcore_axis_name="s"),
        compiler_params=pltpu.CompilerParams(needs_layout_passes=False,
                                             use_tc_tiling_on_sc=True),
        scratch_types=[
            pltpu.VMEM((CHUNK_ROWS, N_COLS), jnp.float32),
            pltpu.VMEM((CHUNK_ROWS, N_COLS), jnp.float32),
            pltpu.VMEM((CHUNK_ROWS,), jnp.int32),
            pltpu.VMEM((CHUNK_ROWS,), jnp.int32),
            pltpu.VMEM((CHUNK_ROWS,), jnp.float32),
            pltpu.VMEM((CHUNK_ROWS,), jnp.float32),
            pltpu.VMEM((CHUNK_ROWS,), jnp.float32),
            pltpu.VMEM((CHUNK_ROWS,), jnp.float32),
            pltpu.VMEM((16,), jnp.float32),
            pltpu.SemaphoreType.DMA,
            pltpu.SemaphoreType.DMA,
        ],
    )
    return kfn(x, lbl, sco, wgt)


def kernel(output, label, score, weight, avg_factor):
    partials = _qfl_partials(output, label.astype(jnp.int32), score, weight)
    return partials.sum() / avg_factor
